# baseline (speedup 1.0000x reference)
import functools

import jax
import jax.numpy as jnp
from jax import lax
from jax.experimental import pallas as pl
from jax.experimental.pallas import tpu as pltpu, tpu_sc as plsc

_B = 2048
_K = 1024
_D = 64
_BB = 512

_NC = 2
_NS = 16
_BPW = _B // (_NC * _NS)


def _dot(a, b):
    return lax.dot_general(a, b, (((1,), (0,)), ((), ())),
                           preferred_element_type=jnp.float32)


def _split(x):
    hi = x.astype(jnp.bfloat16)
    lo = (x - hi.astype(jnp.float32)).astype(jnp.bfloat16)
    return hi, lo


def _idx_block(ze_ref, emb_ref, idx_ref):
    ze = ze_ref[...]
    emb = emb_ref[...]
    ze_hi, ze_lo = _split(ze)
    emb_hi, emb_lo = _split(emb)
    m = _dot(ze_hi, emb_hi) + (_dot(ze_hi, emb_lo) + _dot(ze_lo, emb_hi))
    r = jnp.sum(ze * ze, axis=1, keepdims=True)
    c = jnp.sum(emb * emb, axis=0, keepdims=True)
    dist = r - 2.0 * m + c
    dmin = jnp.min(dist, axis=1, keepdims=True)
    ids = lax.broadcasted_iota(jnp.int32, dist.shape, 1)
    idx_ref[...] = jnp.min(jnp.where(dist == dmin, ids, jnp.int32(_D)),
                           axis=1, keepdims=True)


def _argmin_idx(ze, emb):
    return pl.pallas_call(
        _idx_block,
        grid=(_B // _BB,),
        in_specs=[
            pl.BlockSpec((_BB, _K), lambda i: (i, 0)),
            pl.BlockSpec((_K, _D), lambda i: (0, 0)),
        ],
        out_specs=pl.BlockSpec((_BB, 1), lambda i: (i, 0)),
        out_shape=jax.ShapeDtypeStruct((_B, 1), jnp.int32),
    )(ze, emb)


_CH = 16
_NCHUNK = _BPW // _CH


@functools.partial(
    pl.kernel,
    out_type=jax.ShapeDtypeStruct((_B, _K), jnp.float32),
    mesh=plsc.VectorSubcoreMesh(core_axis_name="c", subcore_axis_name="s"),
    scratch_types=[
        pltpu.VMEM((_BPW,), jnp.int32),
        pltpu.VMEM((2, _CH, _K), jnp.float32),
        [pltpu.SemaphoreType.DMA] * 2,
        [pltpu.SemaphoreType.DMA] * 2,
    ],
)
def _sc_gather(ze_hbm, idx_hbm, out_hbm, idx_v, rows_v, gsems, ssems):
    wid = lax.axis_index("s") * _NC + lax.axis_index("c")
    base = wid * _BPW
    pltpu.sync_copy(idx_hbm.at[pl.ds(base, _BPW)], idx_v)
    gathers = [None, None]
    stores = [None, None]
    for j in range(_NCHUNK):
        s = j % 2
        if stores[s] is not None:
            stores[s].wait()
        gathers[s] = pltpu.async_copy(
            ze_hbm.at[idx_v.at[pl.ds(j * _CH, _CH)]], rows_v.at[s], gsems[s])
        gathers[s].wait()
        stores[s] = pltpu.async_copy(
            rows_v.at[s], out_hbm.at[pl.ds(base + j * _CH, _CH)], ssems[s])
    for s in range(2):
        if stores[s] is not None:
            stores[s].wait()


def kernel(ze, emb):
    idx = _argmin_idx(ze, emb).reshape(_B)
    return _sc_gather(ze, idx)


# speedup vs baseline: 3.6928x; 3.6928x over previous
import jax
import jax.numpy as jnp
from jax import lax
from jax.experimental import pallas as pl

_B = 2048
_K = 1024
_D = 64
_BB = 512


def _dot(a, b):
    return lax.dot_general(a, b, (((1,), (0,)), ((), ())),
                           preferred_element_type=jnp.float32)


def _split(x):
    hi = x.astype(jnp.bfloat16)
    lo = (x - hi.astype(jnp.float32)).astype(jnp.bfloat16)
    return hi, lo


def _vq_block(ze_ref, emb_ref, zetop_ref, out_ref):
    ze = ze_ref[...]
    emb = emb_ref[...]
    ze_hi, ze_lo = _split(ze)
    emb_hi, emb_lo = _split(emb)
    m = _dot(ze_hi, emb_hi) + (_dot(ze_hi, emb_lo) + _dot(ze_lo, emb_hi))
    r = jnp.sum(ze * ze, axis=1, keepdims=True)
    c = jnp.sum(emb * emb, axis=0, keepdims=True)
    dist = r - 2.0 * m + c
    dmin = jnp.min(dist, axis=1, keepdims=True)
    ids = lax.broadcasted_iota(jnp.int32, dist.shape, 1)
    idx = jnp.min(jnp.where(dist == dmin, ids, jnp.int32(_D)),
                  axis=1, keepdims=True)
    onehot = (ids == idx).astype(jnp.float32)
    out_ref[...] = _dot(onehot, zetop_ref[...])


def kernel(ze, emb):
    return pl.pallas_call(
        _vq_block,
        grid=(_B // _BB,),
        in_specs=[
            pl.BlockSpec((_BB, _K), lambda i: (i, 0)),
            pl.BlockSpec((_K, _D), lambda i: (0, 0)),
            pl.BlockSpec((_D, _K), lambda i: (0, 0)),
        ],
        out_specs=pl.BlockSpec((_BB, _K), lambda i: (i, 0)),
        out_shape=jax.ShapeDtypeStruct((_B, _K), jnp.float32),
    )(ze, emb, ze)
